# baseline (speedup 1.0000x reference)
import functools

import jax
import jax.numpy as jnp
from jax import lax
from jax.experimental import pallas as pl
from jax.experimental.pallas import tpu as pltpu
from jax.experimental.pallas import tpu_sc as plsc

_NC = 2
_NS = 16
_NW = _NC * _NS

_B = 1024 * 200
_D = 768
_BPW = _B // _NW
_CHUNK = 32
_NCHUNK = _BPW // _CHUNK
_NBUF = 4

_mesh = plsc.VectorSubcoreMesh(core_axis_name="c", subcore_axis_name="s")


@functools.partial(
    pl.kernel,
    out_type=jax.ShapeDtypeStruct((_B, _D), jnp.float32),
    mesh=_mesh,
    scratch_types=[
        pltpu.VMEM((_NCHUNK, _CHUNK), jnp.int32),
        pltpu.VMEM((_NBUF, _CHUNK, _D), jnp.float32),
        [pltpu.SemaphoreType.DMA] * _NBUF,
        [pltpu.SemaphoreType.DMA] * _NBUF,
    ],
)
def _gather_rows(table_hbm, idx_hbm, out_hbm, idx_v, rows_v, gsem, ssem):
    wid = lax.axis_index("s") * _NC + lax.axis_index("c")
    base = wid * _BPW
    pltpu.sync_copy(idx_hbm.at[wid], idx_v)

    def gather(c, b):
        return pltpu.make_async_copy(
            table_hbm.at[idx_v.at[c]], rows_v.at[b], gsem[b]
        )

    def store(c, b):
        return pltpu.make_async_copy(
            rows_v.at[b], out_hbm.at[pl.ds(base + c * _CHUNK, _CHUNK)], ssem[b]
        )

    gather(0, 0).start()
    gather(1, 1).start()

    @pl.loop(0, _NCHUNK, step=_NBUF)
    def _round(j):
        for b in range(_NBUF):
            c = j + b
            gather(c, b).wait()
            store(c, b).start()

            @pl.when(c >= 2)
            def _():
                store(c - 2, (b - 2) % _NBUF).wait()

            @pl.when(c + 2 < _NCHUNK)
            def _():
                gather(c + 2, (b + 2) % _NBUF).start()

    store(_NCHUNK - 2, (_NCHUNK - 2) % _NBUF).wait()
    store(_NCHUNK - 1, (_NCHUNK - 1) % _NBUF).wait()


def kernel(x, table):
    idx = x.reshape(_NW, _NCHUNK, _CHUNK)
    out = _gather_rows(table, idx)
    return out.reshape(x.shape[0], x.shape[1], _D)
